# baseline (speedup 1.0000x reference)
import functools

import jax
import jax.numpy as jnp
from jax import lax
from jax.experimental import pallas as pl
from jax.experimental.pallas import tpu as pltpu
from jax.experimental.pallas import tpu_sc as plsc

_N = 100000
_D = 128
_NW = 32
_B = 128
_BP = _B // 2
_CHUNKS = 25
_NPAD = _NW * _B * _CHUNKS
_NPAD2 = _NPAD // 2

_HIMASK = -65536
_RNE = 0x7FFF


_BM = 1024
_BM2 = _BM // 2


def _pack_tc(t):
    t3 = t.reshape(t.shape[0] // 2, 2, _D)
    e = lax.bitcast_convert_type(t3[:, 0, :].astype(jnp.bfloat16), jnp.uint16)
    o = lax.bitcast_convert_type(t3[:, 1, :].astype(jnp.bfloat16), jnp.uint16)
    return e.astype(jnp.int32) | (o.astype(jnp.int32) << 16)


def _unpack_tc(w):
    lo = lax.bitcast_convert_type(w << 16, jnp.float32)
    hi = lax.bitcast_convert_type(w & _HIMASK, jnp.float32)
    return jnp.stack([lo, hi], axis=1).reshape(w.shape[0] * 2, _D)


def _mm_tables(x, wc, b):
    y = jnp.dot(x, wc, preferred_element_type=jnp.float32)
    outs = []
    for k in range(5):
        t = y[:, k * _D:(k + 1) * _D]
        if k == 0:
            t = t + b
        outs.append(_pack_tc(t))
    return outs


def _mm_body_f32(x_ref, wc_ref, b_ref, o0, o1, o2, o3, o4):
    outs = _mm_tables(x_ref[...], wc_ref[...], b_ref[...])
    for o, t in zip((o0, o1, o2, o3, o4), outs):
        o[...] = t


def _mm_body_packed(h_ref, wc_ref, b_ref, o0, o1, o2, o3, o4):
    outs = _mm_tables(_unpack_tc(h_ref[...]), wc_ref[...], b_ref[...])
    for o, t in zip((o0, o1, o2, o3, o4), outs):
        o[...] = t


def _tc_tables(xp, wc, b, packed_input):
    grid = _NPAD // _BM
    out_sd = jax.ShapeDtypeStruct((_NPAD2, _D), jnp.int32)
    obs = pl.BlockSpec((_BM2, _D), lambda i: (i, 0))
    in_spec = (pl.BlockSpec((_BM2, _D), lambda i: (i, 0)) if packed_input
               else pl.BlockSpec((_BM, _D), lambda i: (i, 0)))
    return pl.pallas_call(
        _mm_body_packed if packed_input else _mm_body_f32,
        grid=(grid,),
        in_specs=[
            in_spec,
            pl.BlockSpec((_D, 5 * _D), lambda i: (0, 0)),
            pl.BlockSpec((1, _D), lambda i: (0, 0)),
        ],
        out_specs=[obs, obs, obs, obs, obs],
        out_shape=[out_sd] * 5,
    )(xp, wc, b)


def _lohi(w):
    return (plsc.bitcast(w << 16, jnp.float32),
            plsc.bitcast(w & _HIMASK, jnp.float32))


def _rne16(v):
    b = plsc.bitcast(v, jnp.int32)
    return jnp.right_shift(b + _RNE + (jnp.right_shift(b, 16) & 1), 16) & 0xFFFF


def _sc_chunks(refs, final):
    if final:
        (y0_hbm, y1_hbm, y2_hbm, y3_hbm, y4_hbm,
         i0_hbm, i1_hbm, i2_hbm, i3_hbm,
         out_hbm,
         x0_v, x1_v, x2_v, x3_v,
         r0_v, r1_v, r2_v, r3_v,
         acc_v, g0_v, g1_v, g2_v, g3_v, f_v,
         s0, s1, s2, s3) = refs
        f_ref = f_v
    else:
        (y0_hbm, y1_hbm, y2_hbm, y3_hbm, y4_hbm,
         i0_hbm, i1_hbm, i2_hbm, i3_hbm,
         out_hbm,
         x0_v, x1_v, x2_v, x3_v,
         r0_v, r1_v, r2_v, r3_v,
         acc_v, g0_v, g1_v, g2_v, g3_v,
         s0, s1, s2, s3) = refs
        f_ref = None
    ih = (i0_hbm, i1_hbm, i2_hbm, i3_hbm)
    xv = (x0_v, x1_v, x2_v, x3_v)
    rv = (r0_v, r1_v, r2_v, r3_v)
    gv = (g0_v, g1_v, g2_v, g3_v)
    tbl = (y1_hbm, y2_hbm, y3_hbm, y4_hbm)
    sems = (s0, s1, s2, s3)
    wid = lax.axis_index("s") * 2 + lax.axis_index("c")
    tb0 = wid * (_CHUNKS * _B)
    pb0 = wid * (_CHUNKS * _BP)

    def chunk_body(ci, carry):
        tb = tb0 + ci * _B
        pb = pb0 + ci * _BP
        for k in range(4):
            pltpu.sync_copy(ih[k].at[pl.ds(tb, _B)], xv[k].at[pl.ds(0, _B)])
        for k in range(4):
            for s8 in range(_B // 16):
                sl = pl.ds(s8 * 16, 16)
                rv[k][sl] = jnp.right_shift(xv[k][sl], 1)
        descs = [pltpu.async_copy(tbl[k].at[rv[k]], gv[k], sems[k])
                 for k in range(4)]
        pltpu.sync_copy(y0_hbm.at[pl.ds(pb, _BP)], acc_v)
        for d in descs:
            d.wait()

        def pair_body(j, jcarry):
            pe, po = [], []
            for k in range(4):
                v = xv[k][pl.ds(2 * j, 16)]
                pe.append(v[0] & 1)
                po.append(v[1] & 1)
            for c in range(_D // 16):
                sl = pl.ds(c * 16, 16)
                se, so = _lohi(acc_v[j, sl])
                for k in range(4):
                    la, ha = _lohi(gv[k][2 * j, sl])
                    se = se + jnp.where(pe[k] == 1, ha, la)
                    lb, hb = _lohi(gv[k][2 * j + 1, sl])
                    so = so + jnp.where(po[k] == 1, hb, lb)
                se = jnp.maximum(se, 0.0)
                so = jnp.maximum(so, 0.0)
                if final:
                    f_ref[2 * j, sl] = se
                    f_ref[2 * j + 1, sl] = so
                else:
                    acc_v[j, sl] = _rne16(se) | (_rne16(so) << 16)
            return jcarry

        lax.fori_loop(0, _BP, pair_body, 0)
        if final:
            pltpu.sync_copy(f_ref, out_hbm.at[pl.ds(tb, _B)])
        else:
            pltpu.sync_copy(acc_v, out_hbm.at[pl.ds(pb, _BP)])
        return carry

    lax.fori_loop(0, _CHUNKS, chunk_body, 0)


def _sc_scratch(final):
    idx = [pltpu.VMEM((_B + 16,), jnp.int32) for _ in range(4)]
    idx += [pltpu.VMEM((_B,), jnp.int32) for _ in range(4)]
    bufs = [pltpu.VMEM((_BP, _D), jnp.int32)]
    bufs += [pltpu.VMEM((_B, _D), jnp.int32) for _ in range(4)]
    if final:
        bufs += [pltpu.VMEM((_B, _D), jnp.float32)]
    sems = [pltpu.SemaphoreType.DMA for _ in range(4)]
    return idx + bufs + sems


@functools.cache
def _sc_mid_kernel():
    return pl.kernel(
        lambda *refs: _sc_chunks(refs, final=False),
        mesh=plsc.VectorSubcoreMesh(core_axis_name="c", subcore_axis_name="s"),
        out_type=jax.ShapeDtypeStruct((_NPAD2, _D), jnp.int32),
        scratch_types=_sc_scratch(final=False),
        compiler_params=pltpu.CompilerParams(needs_layout_passes=False),
    )


@functools.cache
def _sc_final_kernel():
    return pl.kernel(
        lambda *refs: _sc_chunks(refs, final=True),
        mesh=plsc.VectorSubcoreMesh(core_axis_name="c", subcore_axis_name="s"),
        out_type=jax.ShapeDtypeStruct((_NPAD, _D), jnp.float32),
        scratch_types=_sc_scratch(final=True),
        compiler_params=pltpu.CompilerParams(needs_layout_passes=False),
    )


def kernel(x, neighbors, W0, b0, W1, b1):
    xp = jnp.pad(x, ((0, _NPAD - _N), (0, 0)))
    nb = jnp.pad(neighbors.astype(jnp.int32), ((0, _NPAD - _N), (0, 0)))
    i0 = nb[:, 0]
    i1 = nb[:, 1]
    i2 = nb[:, 2]
    i3 = nb[:, 3]

    def wcat(W):
        return W.reshape(5, _D, _D).transpose(1, 0, 2).reshape(_D, 5 * _D)

    y = _tc_tables(xp, wcat(W0), b0.reshape(1, _D), packed_input=False)
    h1 = _sc_mid_kernel()(*y, i0, i1, i2, i3)
    y = _tc_tables(h1, wcat(W1), b1.reshape(1, _D), packed_input=True)
    out = _sc_final_kernel()(*y, i0, i1, i2, i3)
    return out[:_N]


# speedup vs baseline: 1.3471x; 1.3471x over previous
import functools

import jax
import jax.numpy as jnp
from jax import lax
from jax.experimental import pallas as pl
from jax.experimental.pallas import tpu as pltpu
from jax.experimental.pallas import tpu_sc as plsc

_N = 100000
_D = 128
_H = _D // 2
_NW = 32
_B = 128
_CHUNKS = 25
_NPAD = _NW * _B * _CHUNKS

_HIMASK = -65536
_RNE = 0x7FFF


_BM = 1024


def _pack_tc(t):
    e = lax.bitcast_convert_type(t[:, :_H].astype(jnp.bfloat16), jnp.uint16)
    o = lax.bitcast_convert_type(t[:, _H:].astype(jnp.bfloat16), jnp.uint16)
    w = e.astype(jnp.int32) | (o.astype(jnp.int32) << 16)
    w3 = w.reshape(t.shape[0] // 2, 2, _H)
    return jnp.concatenate([w3[:, 0, :], w3[:, 1, :]], axis=1)


def _unpack_tc(w2):
    w = jnp.stack([w2[:, :_H], w2[:, _H:]], axis=1).reshape(w2.shape[0] * 2, _H)
    lo = lax.bitcast_convert_type(w << 16, jnp.float32)
    hi = lax.bitcast_convert_type(w & _HIMASK, jnp.float32)
    return jnp.concatenate([lo, hi], axis=1)


def _mm_tables(x, wc, b):
    y = jnp.dot(x, wc, preferred_element_type=jnp.float32)
    outs = []
    for k in range(5):
        t = y[:, k * _D:(k + 1) * _D]
        if k == 0:
            t = t + b
        outs.append(_pack_tc(t))
    return outs


def _mm_body_f32(x_ref, wc_ref, b_ref, o0, o1, o2, o3, o4):
    outs = _mm_tables(x_ref[...], wc_ref[...], b_ref[...])
    for o, t in zip((o0, o1, o2, o3, o4), outs):
        o[...] = t


def _mm_body_packed(h_ref, wc_ref, b_ref, o0, o1, o2, o3, o4):
    outs = _mm_tables(_unpack_tc(h_ref[...]), wc_ref[...], b_ref[...])
    for o, t in zip((o0, o1, o2, o3, o4), outs):
        o[...] = t


def _tc_tables(xp, wc, b, packed_input):
    grid = _NPAD // _BM
    out_sd = jax.ShapeDtypeStruct((_NPAD // 2, _D), jnp.int32)
    obs = pl.BlockSpec((_BM // 2, _D), lambda i: (i, 0))
    in_spec = (pl.BlockSpec((_BM // 2, _D), lambda i: (i, 0)) if packed_input
               else pl.BlockSpec((_BM, _D), lambda i: (i, 0)))
    return pl.pallas_call(
        _mm_body_packed if packed_input else _mm_body_f32,
        grid=(grid,),
        in_specs=[
            in_spec,
            pl.BlockSpec((_D, 5 * _D), lambda i: (0, 0)),
            pl.BlockSpec((1, _D), lambda i: (0, 0)),
        ],
        out_specs=[obs, obs, obs, obs, obs],
        out_shape=[out_sd] * 5,
    )(xp, wc, b)


def _halves(ref, r, s):
    w = ref[r, s]
    lo = plsc.bitcast(w << 16, jnp.float32)
    hi = plsc.bitcast(w & _HIMASK, jnp.float32)
    return lo, hi


def _repack(lo, hi):
    lb = plsc.bitcast(lo, jnp.int32)
    hb = plsc.bitcast(hi, jnp.int32)
    lr = lb + _RNE + (jnp.right_shift(lb, 16) & 1)
    hr = hb + _RNE + (jnp.right_shift(hb, 16) & 1)
    return (jnp.right_shift(lr, 16) & 0xFFFF) | (hr & _HIMASK)


def _sc_chunks(refs, final):
    if final:
        (y0_hbm, y1_hbm, y2_hbm, y3_hbm, y4_hbm,
         i0_hbm, i1_hbm, i2_hbm, i3_hbm,
         outl_hbm, outh_hbm,
         i0_v, i1_v, i2_v, i3_v,
         acc_v, g0_v, g1_v, g2_v, g3_v, fl_v, fh_v,
         s0, s1, s2, s3) = refs
    else:
        (y0_hbm, y1_hbm, y2_hbm, y3_hbm, y4_hbm,
         i0_hbm, i1_hbm, i2_hbm, i3_hbm,
         out_hbm,
         i0_v, i1_v, i2_v, i3_v,
         acc_v, g0_v, g1_v, g2_v, g3_v,
         s0, s1, s2, s3) = refs
    wid = lax.axis_index("s") * 2 + lax.axis_index("c")
    base0 = wid * (_CHUNKS * _B)

    def chunk_body(ci, carry):
        base = base0 + ci * _B
        pltpu.sync_copy(i0_hbm.at[pl.ds(base, _B)], i0_v)
        pltpu.sync_copy(i1_hbm.at[pl.ds(base, _B)], i1_v)
        pltpu.sync_copy(i2_hbm.at[pl.ds(base, _B)], i2_v)
        pltpu.sync_copy(i3_hbm.at[pl.ds(base, _B)], i3_v)
        d0 = pltpu.async_copy(y1_hbm.at[i0_v], g0_v, s0)
        d1 = pltpu.async_copy(y2_hbm.at[i1_v], g1_v, s1)
        d2 = pltpu.async_copy(y3_hbm.at[i2_v], g2_v, s2)
        d3 = pltpu.async_copy(y4_hbm.at[i3_v], g3_v, s3)
        pltpu.sync_copy(y0_hbm.at[pl.ds(base, _B)], acc_v)
        d0.wait()
        d1.wait()
        d2.wait()
        d3.wait()

        def row_body(r, rcarry):
            for c in range(_H // 16):
                s = pl.ds(c * 16, 16)
                lo, hi = _halves(acc_v, r, s)
                l0, h0 = _halves(g0_v, r, s)
                l1, h1 = _halves(g1_v, r, s)
                l2, h2 = _halves(g2_v, r, s)
                l3, h3 = _halves(g3_v, r, s)
                lo = jnp.maximum(lo + l0 + l1 + l2 + l3, 0.0)
                hi = jnp.maximum(hi + h0 + h1 + h2 + h3, 0.0)
                if final:
                    fl_v[r, s] = lo
                    fh_v[r, s] = hi
                else:
                    acc_v[r, s] = _repack(lo, hi)
            return rcarry

        lax.fori_loop(0, _B, row_body, 0)
        if final:
            pltpu.sync_copy(fl_v, outl_hbm.at[pl.ds(base, _B)])
            pltpu.sync_copy(fh_v, outh_hbm.at[pl.ds(base, _B)])
        else:
            pltpu.sync_copy(acc_v, out_hbm.at[pl.ds(base, _B)])
        return carry

    lax.fori_loop(0, _CHUNKS, chunk_body, 0)


_SC_SCRATCH = [
    pltpu.VMEM((_B,), jnp.int32),
    pltpu.VMEM((_B,), jnp.int32),
    pltpu.VMEM((_B,), jnp.int32),
    pltpu.VMEM((_B,), jnp.int32),
    pltpu.VMEM((_B, _H), jnp.int32),
    pltpu.VMEM((_B, _H), jnp.int32),
    pltpu.VMEM((_B, _H), jnp.int32),
    pltpu.VMEM((_B, _H), jnp.int32),
    pltpu.VMEM((_B, _H), jnp.int32),
]
_SC_SEMS = [
    pltpu.SemaphoreType.DMA,
    pltpu.SemaphoreType.DMA,
    pltpu.SemaphoreType.DMA,
    pltpu.SemaphoreType.DMA,
]
_SC_PARAMS = pltpu.CompilerParams(
    needs_layout_passes=False, use_tc_tiling_on_sc=False)


@functools.cache
def _sc_mid_kernel():
    return pl.kernel(
        lambda *refs: _sc_chunks(refs, final=False),
        mesh=plsc.VectorSubcoreMesh(core_axis_name="c", subcore_axis_name="s"),
        out_type=jax.ShapeDtypeStruct((_NPAD, _H), jnp.int32),
        scratch_types=_SC_SCRATCH + _SC_SEMS,
        compiler_params=_SC_PARAMS,
    )


@functools.cache
def _sc_final_kernel():
    return pl.kernel(
        lambda *refs: _sc_chunks(refs, final=True),
        mesh=plsc.VectorSubcoreMesh(core_axis_name="c", subcore_axis_name="s"),
        out_type=(jax.ShapeDtypeStruct((_NPAD, _H), jnp.float32),
                  jax.ShapeDtypeStruct((_NPAD, _H), jnp.float32)),
        scratch_types=_SC_SCRATCH + [
            pltpu.VMEM((_B, _H), jnp.float32),
            pltpu.VMEM((_B, _H), jnp.float32),
        ] + _SC_SEMS,
        compiler_params=_SC_PARAMS,
    )


def kernel(x, neighbors, W0, b0, W1, b1):
    xp = jnp.pad(x, ((0, _NPAD - _N), (0, 0)))
    nb = jnp.pad(neighbors.astype(jnp.int32), ((0, _NPAD - _N), (0, 0)))
    i0 = nb[:, 0]
    i1 = nb[:, 1]
    i2 = nb[:, 2]
    i3 = nb[:, 3]

    def wcat(W):
        return W.reshape(5, _D, _D).transpose(1, 0, 2).reshape(_D, 5 * _D)

    def v64(t):
        return t.reshape(_NPAD, _H)

    y = _tc_tables(xp, wcat(W0), b0.reshape(1, _D), packed_input=False)
    h1 = _sc_mid_kernel()(*[v64(t) for t in y], i0, i1, i2, i3)
    y = _tc_tables(h1.reshape(_NPAD // 2, _D), wcat(W1), b1.reshape(1, _D),
                   packed_input=True)
    outl, outh = _sc_final_kernel()(*[v64(t) for t in y], i0, i1, i2, i3)
    return jnp.concatenate([outl, outh], axis=1)[:_N]
